# Initial kernel scaffold; baseline (speedup 1.0000x reference)
import jax
import jax.numpy as jnp
from jax.experimental import pallas as pl


def kernel(idx, props, W1, b1, W2, b2, type_emb, type_index):
    raise NotImplementedError("write your pallas kernel here")


# speedup vs baseline: 2.2847x; 2.2847x over previous
import functools

import jax
import jax.numpy as jnp
from jax.experimental import pallas as pl

_BLK = 1024


def _mlp_block(props_ref, w1_ref, b1_ref, w2_ref, b2_ref, te_ref, out_ref):
    p = props_ref[:, 0:1]
    valid = jnp.logical_not(jnp.isnan(p))
    x = jnp.where(valid, p, 0.0)
    h = x * w1_ref[0, :][None, :] + b1_ref[0, :][None, :]
    g = 0.5 * h * (1.0 + jax.lax.erf(h * 0.7071067811865476))
    out = jnp.dot(g, w2_ref[...], preferred_element_type=jnp.float32)
    out = out + b2_ref[0, :][None, :] + te_ref[0, :][None, :]
    out_ref[...] = jnp.where(valid, out, 0.0)


@functools.partial(jax.jit, static_argnames=())
def kernel(idx, props, W1, b1, W2, b2, type_emb, type_index):
    del idx
    b = props.shape[0]
    two_n = W1.shape[1]
    n = W2.shape[1]
    te_row = jnp.take(type_emb, jnp.asarray(type_index, jnp.int32)[None], axis=0)
    b1_2d = b1.reshape(1, two_n)
    b2_2d = b2.reshape(1, n)

    grid = (b // _BLK,)
    out = pl.pallas_call(
        _mlp_block,
        grid=grid,
        in_specs=[
            pl.BlockSpec((_BLK, 1), lambda i: (i, 0)),
            pl.BlockSpec((1, two_n), lambda i: (0, 0)),
            pl.BlockSpec((1, two_n), lambda i: (0, 0)),
            pl.BlockSpec((two_n, n), lambda i: (0, 0)),
            pl.BlockSpec((1, n), lambda i: (0, 0)),
            pl.BlockSpec((1, n), lambda i: (0, 0)),
        ],
        out_specs=pl.BlockSpec((_BLK, n), lambda i: (i, 0)),
        out_shape=jax.ShapeDtypeStruct((b, n), jnp.float32),
    )(props, W1, b1_2d, W2, b2_2d, te_row)
    return out.reshape(b, 1, n)


# speedup vs baseline: 2.6175x; 1.1457x over previous
import functools

import jax
import jax.numpy as jnp
from jax.experimental import pallas as pl
from jax.experimental.pallas import tpu as pltpu

_BLK = 2048


def _mlp_block(props_ref, w1_ref, b1_ref, w2_ref, c_ref, out_ref):
    p = props_ref[:, 0:1]
    h = p * w1_ref[0, :][None, :] + b1_ref[0, :][None, :]
    g = h + h * jax.lax.erf(h * 0.7071067811865476)
    out = jnp.dot(g, w2_ref[...], preferred_element_type=jnp.float32)
    out = out + c_ref[0, :][None, :]
    valid = jnp.logical_not(jnp.isnan(p))
    out_ref[...] = jnp.where(valid, out, 0.0)


@functools.partial(jax.jit, static_argnames=())
def kernel(idx, props, W1, b1, W2, b2, type_emb, type_index):
    del idx
    b = props.shape[0]
    two_n = W1.shape[1]
    n = W2.shape[1]
    te_row = jnp.take(type_emb, jnp.asarray(type_index, jnp.int32)[None], axis=0)
    c = (b2.reshape(1, n) + te_row)
    w2h = 0.5 * W2
    b1_2d = b1.reshape(1, two_n)

    grid = (b // _BLK,)
    out = pl.pallas_call(
        _mlp_block,
        grid=grid,
        in_specs=[
            pl.BlockSpec((_BLK, 1), lambda i: (i, 0)),
            pl.BlockSpec((1, two_n), lambda i: (0, 0)),
            pl.BlockSpec((1, two_n), lambda i: (0, 0)),
            pl.BlockSpec((two_n, n), lambda i: (0, 0)),
            pl.BlockSpec((1, n), lambda i: (0, 0)),
        ],
        out_specs=pl.BlockSpec((_BLK, n), lambda i: (i, 0)),
        out_shape=jax.ShapeDtypeStruct((b, n), jnp.float32),
        compiler_params=pltpu.CompilerParams(
            dimension_semantics=("parallel",)),
    )(props, W1, b1_2d, w2h, c)
    return out.reshape(b, 1, n)


# speedup vs baseline: 2.8173x; 1.0763x over previous
import functools

import jax
import jax.numpy as jnp
from jax.experimental import pallas as pl
from jax.experimental.pallas import tpu as pltpu

_BLK = 4096


def _mlp_block(props_ref, w1_ref, b1_ref, w2_ref, c_ref, out_ref):
    p = props_ref[:, 0:1]
    h = p * w1_ref[0, :][None, :] + b1_ref[0, :][None, :]
    g = h + h * jax.lax.erf(h * 0.7071067811865476)
    out = jnp.dot(g, w2_ref[...], preferred_element_type=jnp.float32)
    out = out + c_ref[0, :][None, :]
    valid = jnp.logical_not(jnp.isnan(p))
    out_ref[...] = jnp.where(valid, out, 0.0)


@functools.partial(jax.jit, static_argnames=())
def kernel(idx, props, W1, b1, W2, b2, type_emb, type_index):
    del idx
    b = props.shape[0]
    two_n = W1.shape[1]
    n = W2.shape[1]
    te_row = jnp.take(type_emb, jnp.asarray(type_index, jnp.int32)[None], axis=0)
    c = (b2.reshape(1, n) + te_row)
    w2h = 0.5 * W2
    b1_2d = b1.reshape(1, two_n)

    grid = (b // _BLK,)
    out = pl.pallas_call(
        _mlp_block,
        grid=grid,
        in_specs=[
            pl.BlockSpec((_BLK, 1), lambda i: (i, 0)),
            pl.BlockSpec((1, two_n), lambda i: (0, 0)),
            pl.BlockSpec((1, two_n), lambda i: (0, 0)),
            pl.BlockSpec((two_n, n), lambda i: (0, 0)),
            pl.BlockSpec((1, n), lambda i: (0, 0)),
        ],
        out_specs=pl.BlockSpec((_BLK, n), lambda i: (i, 0)),
        out_shape=jax.ShapeDtypeStruct((b, n), jnp.float32),
        compiler_params=pltpu.CompilerParams(
            dimension_semantics=("parallel",)),
    )(props, W1, b1_2d, w2h, c)
    return out.reshape(b, 1, n)
